# Initial kernel scaffold; baseline (speedup 1.0000x reference)
import jax
import jax.numpy as jnp
from jax.experimental import pallas as pl


def kernel(x, Q, A, AX, W_pre, b_pre, bn0_g, bn0_b, gcn0_g, gcn0_b, gcn0_W, gcn0_bias, gcn1_g, gcn1_b, gcn1_W, gcn1_bias, mpnn0_g, mpnn0_b, mpnn0_W, mpnn0_bias, mpnn1_g, mpnn1_b, mpnn1_W, mpnn1_bias, W_out, b_out):
    raise NotImplementedError("write your pallas kernel here")


# speedup vs baseline: 21.3937x; 21.3937x over previous
import functools

import jax
import jax.numpy as jnp
from jax import lax
from jax.experimental import pallas as pl
from jax.experimental.pallas import tpu as pltpu
from jax.experimental.pallas import tpu_sc as plsc

N = 10000
C = 128
NSUP = 1024
NCLASS = 16
NEDGE = 320000

NC, NS, L = 2, 16, 16
NW = NC * NS
BLK = 80
EPT = 10240
NBLK = EPT // BLK
NE_PAD = EPT * NW
NACC = 10240
RPT = NACC // NS
DPT = N // NS

_HI = lax.Precision.HIGHEST
_F32 = jnp.float32


def _bn(x, g, b):
    m = jnp.mean(x, axis=0, keepdims=True)
    v = jnp.mean((x - m) ** 2, axis=0, keepdims=True)
    return (x - m) * lax.rsqrt(v + 1e-5) * g + b


def _l2n(x):
    nn = jnp.sqrt(jnp.sum(x * x, axis=1, keepdims=True))
    return x / jnp.maximum(nn, 1e-12)


def _leaky(x):
    return jnp.where(x >= 0, x, 0.01 * x)


_MESH = plsc.VectorSubcoreMesh(core_axis_name="c", subcore_axis_name="s")


@functools.partial(
    pl.kernel,
    out_type=jax.ShapeDtypeStruct((NC, NACC, C), _F32),
    mesh=_MESH,
    scratch_types=[
        pltpu.VMEM((NBLK, BLK), jnp.int32),
        pltpu.VMEM((BLK, C), _F32),
        pltpu.VMEM((BLK, C), _F32),
        pltpu.VMEM_SHARED((NACC, C), _F32),
    ],
)
def _deg_kernel(dst_hbm, out_hbm, dst_v, one_v, z_v, acc):
    c = lax.axis_index("c")
    s = lax.axis_index("s")
    wid = c * NS + s

    e0 = jnp.where(lax.iota(jnp.int32, L) == 0, 1.0, 0.0).astype(_F32)
    zv = jnp.zeros((L,), _F32)

    def init_row(i, carry):
        one_v[i, pl.ds(0, L)] = e0
        z_v[i, pl.ds(0, L)] = zv
        for j in range(1, C // L):
            one_v[i, pl.ds(j * L, L)] = zv
            z_v[i, pl.ds(j * L, L)] = zv
        return carry

    lax.fori_loop(0, BLK, init_row, 0)

    for k in range(RPT // BLK):
        pltpu.sync_copy(z_v, acc.at[pl.ds(s * RPT + k * BLK, BLK)])

    pltpu.sync_copy(dst_hbm.at[pl.ds(wid * NBLK, NBLK)], dst_v)
    plsc.subcore_barrier()

    def blk_body(b, carry):
        pltpu.sync_copy(one_v, acc.at[dst_v.at[b]], add=True)
        return carry

    lax.fori_loop(0, NBLK, blk_body, 0)

    plsc.subcore_barrier()
    pltpu.sync_copy(acc.at[pl.ds(s * RPT, RPT)],
                    out_hbm.at[c, pl.ds(s * RPT, RPT)])


@functools.partial(
    pl.kernel,
    out_type=jax.ShapeDtypeStruct((NC, NACC, C), _F32),
    mesh=_MESH,
    scratch_types=[
        pltpu.VMEM((EPT,), jnp.int32),
        pltpu.VMEM((NBLK, BLK), jnp.int32),
        pltpu.VMEM((BLK, C), _F32),
        pltpu.VMEM((BLK, C), _F32),
        pltpu.VMEM_SHARED((NACC, C), _F32),
        pltpu.SemaphoreType.DMA,
        pltpu.SemaphoreType.DMA,
    ],
)
def _prop_kernel(g_hbm, src_hbm, dst_hbm, out_hbm,
                 src_v, dst_v, buf0, buf1, acc, sem0, sem1):
    c = lax.axis_index("c")
    s = lax.axis_index("s")
    wid = c * NS + s
    zv = jnp.zeros((L,), _F32)

    def zrow(i, carry):
        for j in range(C // L):
            buf0[i, pl.ds(j * L, L)] = zv
        return carry

    lax.fori_loop(0, BLK, zrow, 0)

    for k in range(RPT // BLK):
        pltpu.sync_copy(buf0, acc.at[pl.ds(s * RPT + k * BLK, BLK)])

    pltpu.sync_copy(src_hbm.at[pl.ds(pl.multiple_of(wid * EPT, EPT), EPT)],
                    src_v)
    pltpu.sync_copy(dst_hbm.at[pl.ds(wid * NBLK, NBLK)], dst_v)
    plsc.subcore_barrier()

    bufs = (buf0, buf1)
    sems = (sem0, sem1)

    def idx_slice(b):
        return src_v.at[pl.ds(pl.multiple_of(b * BLK, BLK), BLK)]

    for j in range(2):
        pltpu.async_copy(g_hbm.at[idx_slice(j)], bufs[j], sems[j])

    def body(i, carry):
        for j in range(2):
            b = i * 2 + j
            pltpu.make_async_copy(g_hbm.at[idx_slice(b)], bufs[j],
                                  sems[j]).wait()
            pltpu.sync_copy(bufs[j], acc.at[dst_v.at[b]], add=True)
            pltpu.async_copy(g_hbm.at[idx_slice(b + 2)], bufs[j], sems[j])
        return carry

    lax.fori_loop(0, NBLK // 2 - 1, body, 0)

    for j in range(2):
        b = NBLK - 2 + j
        pltpu.make_async_copy(g_hbm.at[idx_slice(b)], bufs[j], sems[j]).wait()
        pltpu.sync_copy(bufs[j], acc.at[dst_v.at[b]], add=True)

    plsc.subcore_barrier()
    pltpu.sync_copy(acc.at[pl.ds(s * RPT, RPT)],
                    out_hbm.at[c, pl.ds(s * RPT, RPT)])


def _ka_body(x_ref, wpre, bpre, bn0g, bn0b, m0g, m0b, m0W, deg_ref,
             h_ref, g0_ref, mt0_ref, dinv_ref):
    h = jnp.dot(x_ref[...], wpre[...], precision=_HI) + bpre[...]
    h = _bn(h, bn0g[...], bn0b[...])
    h_ref[...] = h
    t = _bn(_l2n(h), m0g[...], m0b[...])
    mt0 = jnp.dot(t, m0W[...], precision=_HI)
    mt0_ref[...] = mt0
    cnt = deg_ref[0, :N, 0:1] + deg_ref[1, :N, 0:1]
    dinv = jnp.broadcast_to(lax.rsqrt(cnt + 1.0), (N, C))
    dinv_ref[...] = dinv
    g0_ref[...] = mt0 * dinv


_ka_call = pl.pallas_call(
    _ka_body,
    out_shape=[
        jax.ShapeDtypeStruct((N, C), _F32),
        jax.ShapeDtypeStruct((N, C), _F32),
        jax.ShapeDtypeStruct((N, C), _F32),
        jax.ShapeDtypeStruct((N, C), _F32),
    ],
)


NBQ = 10
RB = N // NBQ


def _kgcn_body(q_ref, h_ref, a_ref, g0g, g0b, g0W, g0bi, g1g, g1b, g1W, g1bi,
               H_ref, acc, csum):
    i = pl.program_id(0)

    @pl.when(i == 0)
    def _():
        acc[...] = jnp.zeros_like(acc)
        csum[...] = jnp.zeros_like(csum)

    qb = q_ref[...]
    acc[...] += lax.dot_general(qb, h_ref[...], (((0,), (0,)), ((), ())),
                                precision=_HI)
    csum[...] += lax.dot_general(qb, jnp.ones((RB, 1), _F32),
                                 (((0,), (0,)), ((), ())), precision=_HI)

    @pl.when(i == NBQ - 1)
    def _():
        sp = acc[...] / csum[...]
        r = lax.broadcasted_iota(jnp.int32, (NSUP, NSUP), 0)
        cc = lax.broadcasted_iota(jnp.int32, (NSUP, NSUP), 1)
        ahat = a_ref[...] + jnp.where(r == cc, 1.0, 0.0).astype(_F32)
        da = lax.rsqrt(jnp.sum(ahat, axis=1, keepdims=True))
        H = sp
        for (g, b_, W, bi) in ((g0g, g0b, g0W, g0bi), (g1g, g1b, g1W, g1bi)):
            H = _bn(_l2n(H), g[...], b_[...])
            Z = jnp.dot(H, W[...], precision=_HI) + bi[...]
            H = _leaky(da * jnp.dot(ahat, da * Z, precision=_HI))
        H_ref[...] = H


_kgcn_call = pl.pallas_call(
    _kgcn_body,
    grid=(NBQ,),
    in_specs=[
        pl.BlockSpec((RB, NSUP), lambda i: (i, 0)),
        pl.BlockSpec((RB, C), lambda i: (i, 0)),
        pl.BlockSpec((NSUP, NSUP), lambda i: (0, 0)),
        pl.BlockSpec((1, C), lambda i: (0, 0)),
        pl.BlockSpec((1, C), lambda i: (0, 0)),
        pl.BlockSpec((C, C), lambda i: (0, 0)),
        pl.BlockSpec((1, C), lambda i: (0, 0)),
        pl.BlockSpec((1, C), lambda i: (0, 0)),
        pl.BlockSpec((1, C), lambda i: (0, 0)),
        pl.BlockSpec((C, C), lambda i: (0, 0)),
        pl.BlockSpec((1, C), lambda i: (0, 0)),
    ],
    out_specs=pl.BlockSpec((NSUP, C), lambda i: (0, 0)),
    out_shape=jax.ShapeDtypeStruct((NSUP, C), _F32),
    scratch_shapes=[
        pltpu.VMEM((NSUP, C), _F32),
        pltpu.VMEM((NSUP, 1), _F32),
    ],
)


def _kt1_body(s0, mt0, dinv, b0, m1g, m1b, m1W, g1_ref, mt1_ref):
    dv = dinv[...]
    prop = dv * (s0[0] + s0[1]) + dv * dv * mt0[...]
    m1 = _leaky(prop + b0[...])
    t = _bn(_l2n(m1), m1g[...], m1b[...])
    mt1 = jnp.dot(t, m1W[...], precision=_HI)
    mt1_ref[...] = mt1
    g1_ref[...] = mt1 * dv


_kt1_call = pl.pallas_call(
    _kt1_body,
    grid=(1,),
    in_specs=[
        pl.BlockSpec((2, N, C), lambda i: (0, 0, 0)),
        pl.BlockSpec((N, C), lambda i: (0, 0)),
        pl.BlockSpec((N, C), lambda i: (0, 0)),
        pl.BlockSpec((1, C), lambda i: (0, 0)),
        pl.BlockSpec((1, C), lambda i: (0, 0)),
        pl.BlockSpec((1, C), lambda i: (0, 0)),
        pl.BlockSpec((C, C), lambda i: (0, 0)),
    ],
    out_specs=[
        pl.BlockSpec((N, C), lambda i: (0, 0)),
        pl.BlockSpec((N, C), lambda i: (0, 0)),
    ],
    out_shape=[
        jax.ShapeDtypeStruct((N, C), _F32),
        jax.ShapeDtypeStruct((N, C), _F32),
    ],
)


def _kfin_body(q_ref, H_ref, s1, mt1, dinv, b1, wout, bout, o_ref):
    dv = dinv[...]
    m2 = _leaky(dv * (s1[0] + s1[1]) + dv * dv * mt1[...] + b1[...])
    r = m2 + jnp.dot(q_ref[...], H_ref[...], precision=_HI)
    lg = jnp.dot(r, wout[...], precision=_HI) + bout[...]
    mx = jnp.max(lg, axis=1, keepdims=True)
    e = jnp.exp(lg - mx)
    o_ref[...] = e / jnp.sum(e, axis=1, keepdims=True)


_kfin_call = pl.pallas_call(
    _kfin_body,
    grid=(NBQ,),
    in_specs=[
        pl.BlockSpec((RB, NSUP), lambda i: (i, 0)),
        pl.BlockSpec((NSUP, C), lambda i: (0, 0)),
        pl.BlockSpec((2, RB, C), lambda i: (0, i, 0)),
        pl.BlockSpec((RB, C), lambda i: (i, 0)),
        pl.BlockSpec((RB, C), lambda i: (i, 0)),
        pl.BlockSpec((1, C), lambda i: (0, 0)),
        pl.BlockSpec((C, NCLASS), lambda i: (0, 0)),
        pl.BlockSpec((1, NCLASS), lambda i: (0, 0)),
    ],
    out_specs=pl.BlockSpec((RB, NCLASS), lambda i: (i, 0)),
    out_shape=jax.ShapeDtypeStruct((N, NCLASS), _F32),
)


def kernel(x, Q, A, AX, W_pre, b_pre, bn0_g, bn0_b,
           gcn0_g, gcn0_b, gcn0_W, gcn0_bias,
           gcn1_g, gcn1_b, gcn1_W, gcn1_bias,
           mpnn0_g, mpnn0_b, mpnn0_W, mpnn0_bias,
           mpnn1_g, mpnn1_b, mpnn1_W, mpnn1_bias,
           W_out, b_out):
    r2 = lambda v: v.reshape(1, -1).astype(_F32)

    ax = AX.astype(jnp.int32)
    npad = NE_PAD - NEDGE
    pad = jnp.arange(npad, dtype=jnp.int32)
    srcp = jnp.concatenate([ax[0], (pad * 13) % N])
    dstp = jnp.concatenate([ax[1], N + pad % (NACC - N)])
    dstp = dstp.reshape(NW * NBLK, BLK)

    degp = _deg_kernel(dstp)
    h, g0, mt0, dinv = _ka_call(x, W_pre, r2(b_pre), r2(bn0_g), r2(bn0_b),
                                r2(mpnn0_g), r2(mpnn0_b), mpnn0_W, degp)
    S0 = _prop_kernel(g0, srcp, dstp)
    H = _kgcn_call(Q, h, A, r2(gcn0_g), r2(gcn0_b), gcn0_W, r2(gcn0_bias),
                   r2(gcn1_g), r2(gcn1_b), gcn1_W, r2(gcn1_bias))
    g1, mt1 = _kt1_call(S0, mt0, dinv, r2(mpnn0_bias),
                        r2(mpnn1_g), r2(mpnn1_b), mpnn1_W)
    S1 = _prop_kernel(g1, srcp, dstp)
    return _kfin_call(Q, H, S1, mt1, dinv, r2(mpnn1_bias),
                      W_out, r2(b_out))


# speedup vs baseline: 24.4373x; 1.1423x over previous
import functools

import jax
import jax.numpy as jnp
from jax import lax
from jax.experimental import pallas as pl
from jax.experimental.pallas import tpu as pltpu
from jax.experimental.pallas import tpu_sc as plsc

N = 10000
C = 128
NSUP = 1024
NCLASS = 16
NEDGE = 320000

NC, NS, L = 2, 16, 16
NW = NC * NS
BLK = 80
EPT = 10240
NBLK = EPT // BLK
NE_PAD = EPT * NW
NACC = 10240
RPT = NACC // NS
DPT = N // NS

_HI = lax.Precision.HIGHEST
_F32 = jnp.float32


def _bn(x, g, b):
    m = jnp.mean(x, axis=0, keepdims=True)
    v = jnp.mean((x - m) ** 2, axis=0, keepdims=True)
    return (x - m) * lax.rsqrt(v + 1e-5) * g + b


def _l2n(x):
    nn = jnp.sqrt(jnp.sum(x * x, axis=1, keepdims=True))
    return x / jnp.maximum(nn, 1e-12)


def _leaky(x):
    return jnp.where(x >= 0, x, 0.01 * x)


_MESH = plsc.VectorSubcoreMesh(core_axis_name="c", subcore_axis_name="s")


@functools.partial(
    pl.kernel,
    out_type=jax.ShapeDtypeStruct((NC, NACC, C), _F32),
    mesh=_MESH,
    scratch_types=[
        pltpu.VMEM((NBLK, BLK), jnp.int32),
        pltpu.VMEM((BLK, C), _F32),
        pltpu.VMEM((BLK, C), _F32),
        pltpu.VMEM_SHARED((NACC, C), _F32),
    ],
)
def _deg_kernel(dst_hbm, out_hbm, dst_v, one_v, z_v, acc):
    c = lax.axis_index("c")
    s = lax.axis_index("s")
    wid = c * NS + s

    e0 = jnp.where(lax.iota(jnp.int32, L) == 0, 1.0, 0.0).astype(_F32)
    zv = jnp.zeros((L,), _F32)

    def init_row(i, carry):
        one_v[i, pl.ds(0, L)] = e0
        z_v[i, pl.ds(0, L)] = zv
        for j in range(1, C // L):
            one_v[i, pl.ds(j * L, L)] = zv
            z_v[i, pl.ds(j * L, L)] = zv
        return carry

    lax.fori_loop(0, BLK, init_row, 0)

    for k in range(RPT // BLK):
        pltpu.sync_copy(z_v, acc.at[pl.ds(s * RPT + k * BLK, BLK)])

    pltpu.sync_copy(dst_hbm.at[pl.ds(wid * NBLK, NBLK)], dst_v)
    plsc.subcore_barrier()

    def blk_body(b, carry):
        pltpu.sync_copy(one_v, acc.at[dst_v.at[b]], add=True)
        return carry

    lax.fori_loop(0, NBLK, blk_body, 0)

    plsc.subcore_barrier()
    pltpu.sync_copy(acc.at[pl.ds(s * RPT, RPT)],
                    out_hbm.at[c, pl.ds(s * RPT, RPT)])


@functools.partial(
    pl.kernel,
    out_type=jax.ShapeDtypeStruct((NC, NACC, C), _F32),
    mesh=_MESH,
    scratch_types=[
        pltpu.VMEM((EPT,), jnp.int32),
        pltpu.VMEM((NBLK, BLK), jnp.int32),
        pltpu.VMEM((BLK, C), _F32),
        pltpu.VMEM((BLK, C), _F32),
        pltpu.VMEM_SHARED((NACC, C), _F32),
        pltpu.SemaphoreType.DMA,
        pltpu.SemaphoreType.DMA,
    ],
)
def _prop_kernel(g_hbm, src_hbm, dst_hbm, out_hbm,
                 src_v, dst_v, buf0, buf1, acc, sem0, sem1):
    c = lax.axis_index("c")
    s = lax.axis_index("s")
    wid = c * NS + s
    zv = jnp.zeros((L,), _F32)

    def zrow(i, carry):
        for j in range(C // L):
            buf0[i, pl.ds(j * L, L)] = zv
        return carry

    lax.fori_loop(0, BLK, zrow, 0)

    for k in range(RPT // BLK):
        pltpu.sync_copy(buf0, acc.at[pl.ds(s * RPT + k * BLK, BLK)])

    pltpu.sync_copy(src_hbm.at[pl.ds(pl.multiple_of(wid * EPT, EPT), EPT)],
                    src_v)
    pltpu.sync_copy(dst_hbm.at[pl.ds(wid * NBLK, NBLK)], dst_v)
    plsc.subcore_barrier()

    bufs = (buf0, buf1)
    sems = (sem0, sem1)

    def idx_slice(b):
        return src_v.at[pl.ds(pl.multiple_of(b * BLK, BLK), BLK)]

    for j in range(2):
        pltpu.async_copy(g_hbm.at[idx_slice(j)], bufs[j], sems[j])

    def body(i, carry):
        for j in range(2):
            b = i * 2 + j
            pltpu.make_async_copy(g_hbm.at[idx_slice(b)], bufs[j],
                                  sems[j]).wait()
            pltpu.sync_copy(bufs[j], acc.at[dst_v.at[b]], add=True)
            pltpu.async_copy(g_hbm.at[idx_slice(b + 2)], bufs[j], sems[j])
        return carry

    lax.fori_loop(0, NBLK // 2 - 1, body, 0)

    for j in range(2):
        b = NBLK - 2 + j
        pltpu.make_async_copy(g_hbm.at[idx_slice(b)], bufs[j], sems[j]).wait()
        pltpu.sync_copy(bufs[j], acc.at[dst_v.at[b]], add=True)

    plsc.subcore_barrier()
    pltpu.sync_copy(acc.at[pl.ds(s * RPT, RPT)],
                    out_hbm.at[c, pl.ds(s * RPT, RPT)])


def _ka_body(x_ref, wpre, bpre, bn0g, bn0b, m0g, m0b, m0W,
             h_ref, mt0_ref):
    h = jnp.dot(x_ref[...], wpre[...], precision=_HI) + bpre[...]
    h = _bn(h, bn0g[...], bn0b[...])
    h_ref[...] = h
    t = _bn(_l2n(h), m0g[...], m0b[...])
    mt0_ref[...] = jnp.dot(t, m0W[...], precision=_HI)


_ka_call = pl.pallas_call(
    _ka_body,
    out_shape=[
        jax.ShapeDtypeStruct((N, C), _F32),
        jax.ShapeDtypeStruct((N, C), _F32),
    ],
)


def _kg0_body(mt0, deg_ref, g0_ref, dinv_ref):
    cnt = deg_ref[0, :, 0:1] + deg_ref[1, :, 0:1]
    dinv = jnp.broadcast_to(lax.rsqrt(cnt + 1.0), (N, C))
    dinv_ref[...] = dinv
    g0_ref[...] = mt0[...] * dinv


_kg0_call = pl.pallas_call(
    _kg0_body,
    grid=(1,),
    in_specs=[
        pl.BlockSpec((N, C), lambda i: (0, 0)),
        pl.BlockSpec((2, N, C), lambda i: (0, 0, 0)),
    ],
    out_specs=[
        pl.BlockSpec((N, C), lambda i: (0, 0)),
        pl.BlockSpec((N, C), lambda i: (0, 0)),
    ],
    out_shape=[
        jax.ShapeDtypeStruct((N, C), _F32),
        jax.ShapeDtypeStruct((N, C), _F32),
    ],
)


NBQ = 10
RB = N // NBQ


def _kgcn_body(q_ref, h_ref, a_ref, g0g, g0b, g0W, g0bi, g1g, g1b, g1W, g1bi,
               H_ref, acc, csum):
    i = pl.program_id(0)

    @pl.when(i == 0)
    def _():
        acc[...] = jnp.zeros_like(acc)
        csum[...] = jnp.zeros_like(csum)

    qb = q_ref[...]
    acc[...] += lax.dot_general(qb, h_ref[...], (((0,), (0,)), ((), ())),
                                precision=_HI)
    csum[...] += lax.dot_general(qb, jnp.ones((RB, 1), _F32),
                                 (((0,), (0,)), ((), ())), precision=_HI)

    @pl.when(i == NBQ - 1)
    def _():
        sp = acc[...] / csum[...]
        r = lax.broadcasted_iota(jnp.int32, (NSUP, NSUP), 0)
        cc = lax.broadcasted_iota(jnp.int32, (NSUP, NSUP), 1)
        ahat = a_ref[...] + jnp.where(r == cc, 1.0, 0.0).astype(_F32)
        da = lax.rsqrt(jnp.sum(ahat, axis=1, keepdims=True))
        H = sp
        for (g, b_, W, bi) in ((g0g, g0b, g0W, g0bi), (g1g, g1b, g1W, g1bi)):
            H = _bn(_l2n(H), g[...], b_[...])
            Z = jnp.dot(H, W[...], precision=_HI) + bi[...]
            H = _leaky(da * jnp.dot(ahat, da * Z, precision=_HI))
        H_ref[...] = H


_kgcn_call = pl.pallas_call(
    _kgcn_body,
    grid=(NBQ,),
    in_specs=[
        pl.BlockSpec((RB, NSUP), lambda i: (i, 0)),
        pl.BlockSpec((RB, C), lambda i: (i, 0)),
        pl.BlockSpec((NSUP, NSUP), lambda i: (0, 0)),
        pl.BlockSpec((1, C), lambda i: (0, 0)),
        pl.BlockSpec((1, C), lambda i: (0, 0)),
        pl.BlockSpec((C, C), lambda i: (0, 0)),
        pl.BlockSpec((1, C), lambda i: (0, 0)),
        pl.BlockSpec((1, C), lambda i: (0, 0)),
        pl.BlockSpec((1, C), lambda i: (0, 0)),
        pl.BlockSpec((C, C), lambda i: (0, 0)),
        pl.BlockSpec((1, C), lambda i: (0, 0)),
    ],
    out_specs=pl.BlockSpec((NSUP, C), lambda i: (0, 0)),
    out_shape=jax.ShapeDtypeStruct((NSUP, C), _F32),
    scratch_shapes=[
        pltpu.VMEM((NSUP, C), _F32),
        pltpu.VMEM((NSUP, 1), _F32),
    ],
)


def _kt1_body(s0, mt0, dinv, b0, m1g, m1b, m1W, g1_ref, mt1_ref):
    dv = dinv[...]
    prop = dv * (s0[0] + s0[1]) + dv * dv * mt0[...]
    m1 = _leaky(prop + b0[...])
    t = _bn(_l2n(m1), m1g[...], m1b[...])
    mt1 = jnp.dot(t, m1W[...], precision=_HI)
    mt1_ref[...] = mt1
    g1_ref[...] = mt1 * dv


_kt1_call = pl.pallas_call(
    _kt1_body,
    grid=(1,),
    in_specs=[
        pl.BlockSpec((2, N, C), lambda i: (0, 0, 0)),
        pl.BlockSpec((N, C), lambda i: (0, 0)),
        pl.BlockSpec((N, C), lambda i: (0, 0)),
        pl.BlockSpec((1, C), lambda i: (0, 0)),
        pl.BlockSpec((1, C), lambda i: (0, 0)),
        pl.BlockSpec((1, C), lambda i: (0, 0)),
        pl.BlockSpec((C, C), lambda i: (0, 0)),
    ],
    out_specs=[
        pl.BlockSpec((N, C), lambda i: (0, 0)),
        pl.BlockSpec((N, C), lambda i: (0, 0)),
    ],
    out_shape=[
        jax.ShapeDtypeStruct((N, C), _F32),
        jax.ShapeDtypeStruct((N, C), _F32),
    ],
)


def _kqh_body(q_ref, H_ref, r_ref):
    r_ref[...] = jnp.dot(q_ref[...], H_ref[...], precision=_HI)


_kqh_call = pl.pallas_call(
    _kqh_body,
    grid=(NBQ,),
    in_specs=[
        pl.BlockSpec((RB, NSUP), lambda i: (i, 0)),
        pl.BlockSpec((NSUP, C), lambda i: (0, 0)),
    ],
    out_specs=pl.BlockSpec((RB, C), lambda i: (i, 0)),
    out_shape=jax.ShapeDtypeStruct((N, C), _F32),
)


def _kfin_body(gr_ref, s1, mt1, dinv, b1, wout, bout, o_ref):
    dv = dinv[...]
    m2 = _leaky(dv * (s1[0] + s1[1]) + dv * dv * mt1[...] + b1[...])
    r = m2 + gr_ref[...]
    lg = jnp.dot(r, wout[...], precision=_HI) + bout[...]
    mx = jnp.max(lg, axis=1, keepdims=True)
    e = jnp.exp(lg - mx)
    o_ref[...] = e / jnp.sum(e, axis=1, keepdims=True)


_kfin_call = pl.pallas_call(
    _kfin_body,
    grid=(NBQ,),
    in_specs=[
        pl.BlockSpec((RB, C), lambda i: (i, 0)),
        pl.BlockSpec((2, RB, C), lambda i: (0, i, 0)),
        pl.BlockSpec((RB, C), lambda i: (i, 0)),
        pl.BlockSpec((RB, C), lambda i: (i, 0)),
        pl.BlockSpec((1, C), lambda i: (0, 0)),
        pl.BlockSpec((C, NCLASS), lambda i: (0, 0)),
        pl.BlockSpec((1, NCLASS), lambda i: (0, 0)),
    ],
    out_specs=pl.BlockSpec((RB, NCLASS), lambda i: (i, 0)),
    out_shape=jax.ShapeDtypeStruct((N, NCLASS), _F32),
)


def kernel(x, Q, A, AX, W_pre, b_pre, bn0_g, bn0_b,
           gcn0_g, gcn0_b, gcn0_W, gcn0_bias,
           gcn1_g, gcn1_b, gcn1_W, gcn1_bias,
           mpnn0_g, mpnn0_b, mpnn0_W, mpnn0_bias,
           mpnn1_g, mpnn1_b, mpnn1_W, mpnn1_bias,
           W_out, b_out):
    r2 = lambda v: v.reshape(1, -1).astype(_F32)

    ax = AX.astype(jnp.int32)
    npad = NE_PAD - NEDGE
    pad = jnp.arange(npad, dtype=jnp.int32)
    srcp = jnp.concatenate([ax[0], (pad * 13) % N])
    dstp = jnp.concatenate([ax[1], N + pad % (NACC - N)])
    dstp = dstp.reshape(NW * NBLK, BLK)

    degp = _deg_kernel(dstp)
    h, mt0 = _ka_call(x, W_pre, r2(b_pre), r2(bn0_g), r2(bn0_b),
                      r2(mpnn0_g), r2(mpnn0_b), mpnn0_W)
    g0, dinv = _kg0_call(mt0, degp)
    S0 = _prop_kernel(g0, srcp, dstp)
    H = _kgcn_call(Q, h, A, r2(gcn0_g), r2(gcn0_b), gcn0_W, r2(gcn0_bias),
                   r2(gcn1_g), r2(gcn1_b), gcn1_W, r2(gcn1_bias))
    g1, mt1 = _kt1_call(S0, mt0, dinv, r2(mpnn0_bias),
                        r2(mpnn1_g), r2(mpnn1_b), mpnn1_W)
    S1 = _prop_kernel(g1, srcp, dstp)
    gcn_result = _kqh_call(Q, H)
    return _kfin_call(gcn_result, S1, mt1, dinv, r2(mpnn1_bias),
                      W_out, r2(b_out))


# speedup vs baseline: 27.4442x; 1.1230x over previous
import functools

import jax
import jax.numpy as jnp
from jax import lax
from jax.experimental import pallas as pl
from jax.experimental.pallas import tpu as pltpu
from jax.experimental.pallas import tpu_sc as plsc

N = 10000
C = 128
NSUP = 1024
NCLASS = 16
NEDGE = 320000

NC, NS, L = 2, 16, 16
NW = NC * NS
BLK = 80
EPT = 10240
NBLK = EPT // BLK
NE_PAD = EPT * NW
NACC = 10240
RPT = NACC // NS
DPT = N // NS

_HI = lax.Precision.HIGHEST
_F32 = jnp.float32


def _bn(x, g, b):
    m = jnp.mean(x, axis=0, keepdims=True)
    v = jnp.mean((x - m) ** 2, axis=0, keepdims=True)
    return (x - m) * lax.rsqrt(v + 1e-5) * g + b


def _l2n(x):
    nn = jnp.sqrt(jnp.sum(x * x, axis=1, keepdims=True))
    return x / jnp.maximum(nn, 1e-12)


def _leaky(x):
    return jnp.where(x >= 0, x, 0.01 * x)


_MESH = plsc.VectorSubcoreMesh(core_axis_name="c", subcore_axis_name="s")


@functools.partial(
    pl.kernel,
    out_type=jax.ShapeDtypeStruct((NC, NACC), _F32),
    mesh=_MESH,
    scratch_types=[
        pltpu.VMEM((NBLK, BLK), jnp.int32),
        pltpu.VMEM((BLK,), _F32),
        pltpu.VMEM((RPT,), _F32),
        pltpu.VMEM_SHARED((NACC,), _F32),
    ],
)
def _deg_kernel(dst_hbm, out_hbm, dst_v, one_v, z_v, acc):
    c = lax.axis_index("c")
    s = lax.axis_index("s")
    wid = c * NS + s

    onev = jnp.ones((L,), _F32)
    zv = jnp.zeros((L,), _F32)

    def init1(i, carry):
        one_v[pl.ds(i * L, L)] = onev
        return carry

    lax.fori_loop(0, BLK // L, init1, 0)

    def init2(i, carry):
        z_v[pl.ds(i * L, L)] = zv
        return carry

    lax.fori_loop(0, RPT // L, init2, 0)

    pltpu.sync_copy(z_v, acc.at[pl.ds(s * RPT, RPT)])
    pltpu.sync_copy(dst_hbm.at[pl.ds(wid * NBLK, NBLK)], dst_v)
    plsc.subcore_barrier()

    def blk_body(b, carry):
        pltpu.sync_copy(one_v, acc.at[dst_v.at[b]], add=True)
        return carry

    lax.fori_loop(0, NBLK, blk_body, 0)

    plsc.subcore_barrier()
    pltpu.sync_copy(acc.at[pl.ds(s * RPT, RPT)],
                    out_hbm.at[c, pl.ds(s * RPT, RPT)])


@functools.partial(
    pl.kernel,
    out_type=jax.ShapeDtypeStruct((NC, NACC, C), _F32),
    mesh=_MESH,
    scratch_types=[
        pltpu.VMEM((EPT,), jnp.int32),
        pltpu.VMEM((NBLK, BLK), jnp.int32),
        pltpu.VMEM((BLK, C), _F32),
        pltpu.VMEM((BLK, C), _F32),
        pltpu.VMEM_SHARED((NACC, C), _F32),
        pltpu.SemaphoreType.DMA,
        pltpu.SemaphoreType.DMA,
    ],
)
def _prop_kernel(g_hbm, src_hbm, dst_hbm, out_hbm,
                 src_v, dst_v, buf0, buf1, acc, sem0, sem1):
    c = lax.axis_index("c")
    s = lax.axis_index("s")
    wid = c * NS + s
    zv = jnp.zeros((L,), _F32)

    def zrow(i, carry):
        for j in range(C // L):
            buf0[i, pl.ds(j * L, L)] = zv
        return carry

    lax.fori_loop(0, BLK, zrow, 0)

    for k in range(RPT // BLK):
        pltpu.sync_copy(buf0, acc.at[pl.ds(s * RPT + k * BLK, BLK)])

    pltpu.sync_copy(src_hbm.at[pl.ds(pl.multiple_of(wid * EPT, EPT), EPT)],
                    src_v)
    pltpu.sync_copy(dst_hbm.at[pl.ds(wid * NBLK, NBLK)], dst_v)
    plsc.subcore_barrier()

    bufs = (buf0, buf1)
    sems = (sem0, sem1)

    def idx_slice(b):
        return src_v.at[pl.ds(pl.multiple_of(b * BLK, BLK), BLK)]

    for j in range(2):
        pltpu.async_copy(g_hbm.at[idx_slice(j)], bufs[j], sems[j])

    def body(i, carry):
        for j in range(2):
            b = i * 2 + j
            pltpu.make_async_copy(g_hbm.at[idx_slice(b)], bufs[j],
                                  sems[j]).wait()
            pltpu.sync_copy(bufs[j], acc.at[dst_v.at[b]], add=True)
            pltpu.async_copy(g_hbm.at[idx_slice(b + 2)], bufs[j], sems[j])
        return carry

    lax.fori_loop(0, NBLK // 2 - 1, body, 0)

    for j in range(2):
        b = NBLK - 2 + j
        pltpu.make_async_copy(g_hbm.at[idx_slice(b)], bufs[j], sems[j]).wait()
        pltpu.sync_copy(bufs[j], acc.at[dst_v.at[b]], add=True)

    plsc.subcore_barrier()
    pltpu.sync_copy(acc.at[pl.ds(s * RPT, RPT)],
                    out_hbm.at[c, pl.ds(s * RPT, RPT)])


def _ka_body(x_ref, wpre, bpre, bn0g, bn0b, m0g, m0b, m0W,
             h_ref, mt0_ref):
    h = jnp.dot(x_ref[...], wpre[...], precision=_HI) + bpre[...]
    h = _bn(h, bn0g[...], bn0b[...])
    h_ref[...] = h
    t = _bn(_l2n(h), m0g[...], m0b[...])
    mt0_ref[...] = jnp.dot(t, m0W[...], precision=_HI)


_ka_call = pl.pallas_call(
    _ka_body,
    out_shape=[
        jax.ShapeDtypeStruct((N, C), _F32),
        jax.ShapeDtypeStruct((N, C), _F32),
    ],
)


def _kg0_body(mt0, deg_ref, g0_ref, dinv_ref):
    cnt = deg_ref[0, :] + deg_ref[1, :]
    col = jnp.reshape(cnt, (NACC, 1))[:N]
    dinv = jnp.broadcast_to(lax.rsqrt(col + 1.0), (N, C))
    dinv_ref[...] = dinv
    g0_ref[...] = mt0[...] * dinv


_kg0_call = pl.pallas_call(
    _kg0_body,
    grid=(1,),
    in_specs=[
        pl.BlockSpec((N, C), lambda i: (0, 0)),
        pl.BlockSpec((2, NACC), lambda i: (0, 0)),
    ],
    out_specs=[
        pl.BlockSpec((N, C), lambda i: (0, 0)),
        pl.BlockSpec((N, C), lambda i: (0, 0)),
    ],
    out_shape=[
        jax.ShapeDtypeStruct((N, C), _F32),
        jax.ShapeDtypeStruct((N, C), _F32),
    ],
)


NBQ = 10
RB = N // NBQ


def _kgcn_body(q_ref, h_ref, a_ref, g0g, g0b, g0W, g0bi, g1g, g1b, g1W, g1bi,
               H_ref, acc, csum):
    i = pl.program_id(0)

    @pl.when(i == 0)
    def _():
        acc[...] = jnp.zeros_like(acc)
        csum[...] = jnp.zeros_like(csum)

    qb = q_ref[...]
    acc[...] += lax.dot_general(qb, h_ref[...], (((0,), (0,)), ((), ())),
                                precision=_HI)
    csum[...] += lax.dot_general(qb, jnp.ones((RB, 1), _F32),
                                 (((0,), (0,)), ((), ())), precision=_HI)

    @pl.when(i == NBQ - 1)
    def _():
        sp = acc[...] / csum[...]
        r = lax.broadcasted_iota(jnp.int32, (NSUP, NSUP), 0)
        cc = lax.broadcasted_iota(jnp.int32, (NSUP, NSUP), 1)
        ahat = a_ref[...] + jnp.where(r == cc, 1.0, 0.0).astype(_F32)
        da = lax.rsqrt(jnp.sum(ahat, axis=1, keepdims=True))
        H = sp
        for (g, b_, W, bi) in ((g0g, g0b, g0W, g0bi), (g1g, g1b, g1W, g1bi)):
            H = _bn(_l2n(H), g[...], b_[...])
            Z = jnp.dot(H, W[...], precision=_HI) + bi[...]
            H = _leaky(da * jnp.dot(ahat, da * Z, precision=_HI))
        H_ref[...] = H


_kgcn_call = pl.pallas_call(
    _kgcn_body,
    grid=(NBQ,),
    in_specs=[
        pl.BlockSpec((RB, NSUP), lambda i: (i, 0)),
        pl.BlockSpec((RB, C), lambda i: (i, 0)),
        pl.BlockSpec((NSUP, NSUP), lambda i: (0, 0)),
        pl.BlockSpec((1, C), lambda i: (0, 0)),
        pl.BlockSpec((1, C), lambda i: (0, 0)),
        pl.BlockSpec((C, C), lambda i: (0, 0)),
        pl.BlockSpec((1, C), lambda i: (0, 0)),
        pl.BlockSpec((1, C), lambda i: (0, 0)),
        pl.BlockSpec((1, C), lambda i: (0, 0)),
        pl.BlockSpec((C, C), lambda i: (0, 0)),
        pl.BlockSpec((1, C), lambda i: (0, 0)),
    ],
    out_specs=pl.BlockSpec((NSUP, C), lambda i: (0, 0)),
    out_shape=jax.ShapeDtypeStruct((NSUP, C), _F32),
    scratch_shapes=[
        pltpu.VMEM((NSUP, C), _F32),
        pltpu.VMEM((NSUP, 1), _F32),
    ],
)


def _kt1_body(s0, mt0, dinv, b0, m1g, m1b, m1W, g1_ref, mt1_ref):
    dv = dinv[...]
    prop = dv * (s0[0] + s0[1]) + dv * dv * mt0[...]
    m1 = _leaky(prop + b0[...])
    t = _bn(_l2n(m1), m1g[...], m1b[...])
    mt1 = jnp.dot(t, m1W[...], precision=_HI)
    mt1_ref[...] = mt1
    g1_ref[...] = mt1 * dv


_kt1_call = pl.pallas_call(
    _kt1_body,
    grid=(1,),
    in_specs=[
        pl.BlockSpec((2, N, C), lambda i: (0, 0, 0)),
        pl.BlockSpec((N, C), lambda i: (0, 0)),
        pl.BlockSpec((N, C), lambda i: (0, 0)),
        pl.BlockSpec((1, C), lambda i: (0, 0)),
        pl.BlockSpec((1, C), lambda i: (0, 0)),
        pl.BlockSpec((1, C), lambda i: (0, 0)),
        pl.BlockSpec((C, C), lambda i: (0, 0)),
    ],
    out_specs=[
        pl.BlockSpec((N, C), lambda i: (0, 0)),
        pl.BlockSpec((N, C), lambda i: (0, 0)),
    ],
    out_shape=[
        jax.ShapeDtypeStruct((N, C), _F32),
        jax.ShapeDtypeStruct((N, C), _F32),
    ],
)


def _kqh_body(q_ref, H_ref, r_ref):
    r_ref[...] = jnp.dot(q_ref[...], H_ref[...], precision=_HI)


_kqh_call = pl.pallas_call(
    _kqh_body,
    grid=(NBQ,),
    in_specs=[
        pl.BlockSpec((RB, NSUP), lambda i: (i, 0)),
        pl.BlockSpec((NSUP, C), lambda i: (0, 0)),
    ],
    out_specs=pl.BlockSpec((RB, C), lambda i: (i, 0)),
    out_shape=jax.ShapeDtypeStruct((N, C), _F32),
)


def _kfin_body(gr_ref, s1, mt1, dinv, b1, wout, bout, o_ref):
    dv = dinv[...]
    m2 = _leaky(dv * (s1[0] + s1[1]) + dv * dv * mt1[...] + b1[...])
    r = m2 + gr_ref[...]
    lg = jnp.dot(r, wout[...], precision=_HI) + bout[...]
    mx = jnp.max(lg, axis=1, keepdims=True)
    e = jnp.exp(lg - mx)
    o_ref[...] = e / jnp.sum(e, axis=1, keepdims=True)


_kfin_call = pl.pallas_call(
    _kfin_body,
    grid=(NBQ,),
    in_specs=[
        pl.BlockSpec((RB, C), lambda i: (i, 0)),
        pl.BlockSpec((2, RB, C), lambda i: (0, i, 0)),
        pl.BlockSpec((RB, C), lambda i: (i, 0)),
        pl.BlockSpec((RB, C), lambda i: (i, 0)),
        pl.BlockSpec((1, C), lambda i: (0, 0)),
        pl.BlockSpec((C, NCLASS), lambda i: (0, 0)),
        pl.BlockSpec((1, NCLASS), lambda i: (0, 0)),
    ],
    out_specs=pl.BlockSpec((RB, NCLASS), lambda i: (i, 0)),
    out_shape=jax.ShapeDtypeStruct((N, NCLASS), _F32),
)


def kernel(x, Q, A, AX, W_pre, b_pre, bn0_g, bn0_b,
           gcn0_g, gcn0_b, gcn0_W, gcn0_bias,
           gcn1_g, gcn1_b, gcn1_W, gcn1_bias,
           mpnn0_g, mpnn0_b, mpnn0_W, mpnn0_bias,
           mpnn1_g, mpnn1_b, mpnn1_W, mpnn1_bias,
           W_out, b_out):
    r2 = lambda v: v.reshape(1, -1).astype(_F32)

    ax = AX.astype(jnp.int32)
    npad = NE_PAD - NEDGE
    pad = jnp.arange(npad, dtype=jnp.int32)
    srcp = jnp.concatenate([ax[0], (pad * 13) % N])
    dstp = jnp.concatenate([ax[1], N + pad % (NACC - N)])
    dstp = dstp.reshape(NW * NBLK, BLK)

    degp = _deg_kernel(dstp)
    h, mt0 = _ka_call(x, W_pre, r2(b_pre), r2(bn0_g), r2(bn0_b),
                      r2(mpnn0_g), r2(mpnn0_b), mpnn0_W)
    g0, dinv = _kg0_call(mt0, degp)
    S0 = _prop_kernel(g0, srcp, dstp)
    H = _kgcn_call(Q, h, A, r2(gcn0_g), r2(gcn0_b), gcn0_W, r2(gcn0_bias),
                   r2(gcn1_g), r2(gcn1_b), gcn1_W, r2(gcn1_bias))
    g1, mt1 = _kt1_call(S0, mt0, dinv, r2(mpnn0_bias),
                        r2(mpnn1_g), r2(mpnn1_b), mpnn1_W)
    S1 = _prop_kernel(g1, srcp, dstp)
    gcn_result = _kqh_call(Q, H)
    return _kfin_call(gcn_result, S1, mt1, dinv, r2(mpnn1_bias),
                      W_out, r2(b_out))
